# baseline (speedup 1.0000x reference)
import functools

import jax
import jax.numpy as jnp
from jax import lax
from jax.experimental import pallas as pl
from jax.experimental.pallas import tpu as pltpu
from jax.experimental.pallas import tpu_sc as plsc

N = 10000
NH = 32
D = 128
LANES = 16
VPR = D // LANES

NC = 2
NS = 16
NW = NC * NS
NPAD = 10240
NPW = NPAD // NW

GROUP = 2
NGRP = NPW // GROUP
NBUF = 2
NCHUNK = 8
CGRP = NGRP // NCHUNK
WAVES = CGRP // NBUF


def _mm_body(x_ref, w_ref, b_ref, zc_ref, zn_ref):
    y = jnp.dot(x_ref[...], w_ref[...], preferred_element_type=jnp.float32)
    zc_ref[...] = y[:, :D] + b_ref[...]
    zn_ref[...] = y[:, D:]


def _matmuls(xpad, w, bias):
    blk = 1024
    return pl.pallas_call(
        _mm_body,
        grid=(NPAD // blk,),
        in_specs=[
            pl.BlockSpec((blk, D), lambda i: (i, 0)),
            pl.BlockSpec((D, 2 * D), lambda i: (0, 0)),
            pl.BlockSpec((1, D), lambda i: (0, 0)),
        ],
        out_specs=[
            pl.BlockSpec((blk, D), lambda i: (i, 0)),
            pl.BlockSpec((blk, D), lambda i: (i, 0)),
        ],
        out_shape=[
            jax.ShapeDtypeStruct((NPAD, D), jnp.float32),
            jax.ShapeDtypeStruct((NPAD, D), jnp.float32),
        ],
    )(xpad, w, bias.reshape(1, D))


def _accum_node(rows, j, zc_v, out_v, ln):
    def row(r, accs):
        return tuple(accs[v] + rows[j * NH + r, pl.ds(LANES * v, LANES)]
                     for v in range(VPR))

    accs = lax.fori_loop(
        0, NH, row,
        tuple(zc_v[ln, pl.ds(LANES * v, LANES)] for v in range(VPR)),
        unroll=8)
    for v in range(VPR):
        out_v[ln, pl.ds(LANES * v, LANES)] = jnp.maximum(
            accs[v], jnp.float32(0.0))


def _agg_body(zn_hbm, zc_hbm, idx_hbm, out_hbm,
              idx_v, rows_v, zc_v, out_v, zn_sh, sems):
    sid = lax.axis_index("s")
    wid = sid * NC + lax.axis_index("c")
    base_n = wid * NPW
    base_g = wid * NGRP

    stage = NPAD // NS
    pltpu.sync_copy(zn_hbm.at[pl.ds(sid * stage, stage)],
                    zn_sh.at[pl.ds(sid * stage, stage)])

    pltpu.sync_copy(idx_hbm.at[pl.ds(base_g, NGRP)], idx_v)
    plsc.subcore_barrier()

    def gather(g, b):
        return pltpu.async_copy(
            zn_sh.at[idx_v.at[g]], rows_v.at[b], sems.at[b])

    for b in range(NBUF):
        gather(b, b)

    for c in range(NCHUNK):
        pltpu.sync_copy(zc_hbm.at[pl.ds(base_n + c * CGRP * GROUP,
                                        CGRP * GROUP)], zc_v)

        def wave(w, carry):
            for b in range(NBUF):
                gl = w * NBUF + b
                g = c * CGRP + gl
                pltpu.make_async_copy(
                    zn_sh.at[idx_v.at[g]], rows_v.at[b], sems.at[b]).wait()
                for j in range(GROUP):
                    _accum_node(rows_v.at[b], j, zc_v, out_v,
                                gl * GROUP + j)

                @pl.when(g + NBUF < NGRP)
                def _():
                    gather(g + NBUF, b)
            return carry

        lax.fori_loop(0, WAVES, wave, 0)

        pltpu.sync_copy(
            out_v, out_hbm.at[pl.ds(base_n + c * CGRP * GROUP,
                                    CGRP * GROUP)])


@functools.partial(
    pl.kernel,
    out_type=jax.ShapeDtypeStruct((NPAD, D), jnp.float32),
    mesh=plsc.VectorSubcoreMesh(core_axis_name="c", subcore_axis_name="s"),
    scratch_types=[
        pltpu.VMEM((NGRP, GROUP * NH), jnp.int32),
        pltpu.VMEM((NBUF, GROUP * NH, D), jnp.float32),
        pltpu.VMEM((CGRP * GROUP, D), jnp.float32),
        pltpu.VMEM((CGRP * GROUP, D), jnp.float32),
        pltpu.VMEM_SHARED((NPAD, D), jnp.float32),
        pltpu.SemaphoreType.DMA((NBUF,)),
    ],
)
def _aggregate(zn_hbm, zc_hbm, idx_hbm, out_hbm,
               idx_v, rows_v, zc_v, out_v, zn_sh, sems):
    _agg_body(zn_hbm, zc_hbm, idx_hbm, out_hbm,
              idx_v, rows_v, zc_v, out_v, zn_sh, sems)


def kernel(vertex, nh_indices, center_weight, nh_weight, bias):
    xpad = jnp.zeros((NPAD, D), jnp.float32).at[:N].set(vertex)
    idx = jnp.zeros((NPAD, NH), jnp.int32).at[:N].set(
        nh_indices.astype(jnp.int32))
    idx_g = idx.reshape(NPAD // GROUP, GROUP * NH)
    w = jnp.concatenate(
        [center_weight, nh_weight * jnp.float32(1.0 / NH)], axis=1)
    zc, zn = _matmuls(xpad, w, bias)
    out = _aggregate(zn, zc, idx_g)
    return out[:N]


# speedup vs baseline: 1.0220x; 1.0220x over previous
import functools

import jax
import jax.numpy as jnp
from jax import lax
from jax.experimental import pallas as pl
from jax.experimental.pallas import tpu as pltpu
from jax.experimental.pallas import tpu_sc as plsc

N = 10000
NH = 32
D = 128
LANES = 16
VPR = D // LANES

NC = 2
NS = 16
NW = NC * NS
NPAD = 10240
NPW = NPAD // NW

GROUP = 2
NGRP = NPW // GROUP
NBUF = 2
NCHUNK = 8
CGRP = NGRP // NCHUNK
WAVES = CGRP // NBUF


def _mm_body(x_ref, w_ref, b_ref, zc_ref, zn_ref):
    y = jnp.dot(x_ref[...], w_ref[...], preferred_element_type=jnp.float32)
    zc_ref[...] = y[:, :D] + b_ref[...]
    zn_ref[...] = y[:, D:]


def _matmuls(xpad, w, bias):
    blk = 1024
    return pl.pallas_call(
        _mm_body,
        grid=(NPAD // blk,),
        in_specs=[
            pl.BlockSpec((blk, D), lambda i: (i, 0)),
            pl.BlockSpec((D, 2 * D), lambda i: (0, 0)),
            pl.BlockSpec((1, D), lambda i: (0, 0)),
        ],
        out_specs=[
            pl.BlockSpec((blk, D), lambda i: (i, 0)),
            pl.BlockSpec((blk, D), lambda i: (i, 0)),
        ],
        out_shape=[
            jax.ShapeDtypeStruct((NPAD, D), jnp.float32),
            jax.ShapeDtypeStruct((NPAD, D), jnp.float32),
        ],
    )(xpad, w, bias.reshape(1, D))


def _accum_node(rows, j, zc_v, out_v, ln):
    def row(r, accs):
        return tuple(accs[v] + rows[j * NH + r, pl.ds(LANES * v, LANES)]
                     for v in range(VPR))

    accs = lax.fori_loop(
        0, NH, row,
        tuple(zc_v[ln, pl.ds(LANES * v, LANES)] for v in range(VPR)),
        unroll=4)
    for v in range(VPR):
        out_v[ln, pl.ds(LANES * v, LANES)] = jnp.maximum(
            accs[v], jnp.float32(0.0))


def _agg_body(zn_hbm, zc_hbm, idx_hbm, out_hbm,
              idx_v, rows_v, zc_v, out_v, zn_sh, sems):
    sid = lax.axis_index("s")
    wid = sid * NC + lax.axis_index("c")
    base_n = wid * NPW
    base_g = wid * NGRP

    stage = NPAD // NS
    pltpu.sync_copy(zn_hbm.at[pl.ds(sid * stage, stage)],
                    zn_sh.at[pl.ds(sid * stage, stage)])

    pltpu.sync_copy(idx_hbm.at[pl.ds(base_g, NGRP)], idx_v)
    plsc.subcore_barrier()

    def gather(g, b):
        return pltpu.async_copy(
            zn_sh.at[idx_v.at[g]], rows_v.at[b], sems.at[b])

    for b in range(NBUF):
        gather(b, b)

    for c in range(NCHUNK):
        pltpu.sync_copy(zc_hbm.at[pl.ds(base_n + c * CGRP * GROUP,
                                        CGRP * GROUP)], zc_v)

        def wave(w, carry):
            for b in range(NBUF):
                gl = w * NBUF + b
                g = c * CGRP + gl
                pltpu.make_async_copy(
                    zn_sh.at[idx_v.at[g]], rows_v.at[b], sems.at[b]).wait()
                for j in range(GROUP):
                    _accum_node(rows_v.at[b], j, zc_v, out_v,
                                gl * GROUP + j)

                @pl.when(g + NBUF < NGRP)
                def _():
                    gather(g + NBUF, b)
            return carry

        lax.fori_loop(0, WAVES, wave, 0)

        pltpu.sync_copy(
            out_v, out_hbm.at[pl.ds(base_n + c * CGRP * GROUP,
                                    CGRP * GROUP)])


@functools.partial(
    pl.kernel,
    out_type=jax.ShapeDtypeStruct((NPAD, D), jnp.float32),
    mesh=plsc.VectorSubcoreMesh(core_axis_name="c", subcore_axis_name="s"),
    scratch_types=[
        pltpu.VMEM((NGRP, GROUP * NH), jnp.int32),
        pltpu.VMEM((NBUF, GROUP * NH, D), jnp.float32),
        pltpu.VMEM((CGRP * GROUP, D), jnp.float32),
        pltpu.VMEM((CGRP * GROUP, D), jnp.float32),
        pltpu.VMEM_SHARED((NPAD, D), jnp.float32),
        pltpu.SemaphoreType.DMA((NBUF,)),
    ],
)
def _aggregate(zn_hbm, zc_hbm, idx_hbm, out_hbm,
               idx_v, rows_v, zc_v, out_v, zn_sh, sems):
    _agg_body(zn_hbm, zc_hbm, idx_hbm, out_hbm,
              idx_v, rows_v, zc_v, out_v, zn_sh, sems)


def kernel(vertex, nh_indices, center_weight, nh_weight, bias):
    xpad = jnp.zeros((NPAD, D), jnp.float32).at[:N].set(vertex)
    idx = jnp.zeros((NPAD, NH), jnp.int32).at[:N].set(
        nh_indices.astype(jnp.int32))
    idx_g = idx.reshape(NPAD // GROUP, GROUP * NH)
    w = jnp.concatenate(
        [center_weight, nh_weight * jnp.float32(1.0 / NH)], axis=1)
    zc, zn = _matmuls(xpad, w, bias)
    out = _aggregate(zn, zc, idx_g)
    return out[:N]


# speedup vs baseline: 1.1832x; 1.1577x over previous
import functools

import jax
import jax.numpy as jnp
from jax import lax
from jax.experimental import pallas as pl
from jax.experimental.pallas import tpu as pltpu
from jax.experimental.pallas import tpu_sc as plsc

N = 10000
NH = 32
D = 128
LANES = 16
VPR = D // LANES

NC = 2
NS = 16
NW = NC * NS
NPAD = 10240
NPW = NPAD // NW
SROWS = 624

GROUP = 2
NGRP = NPW // GROUP
NBUF = 2
NCHUNK = 4
CGRP = NGRP // NCHUNK
WAVES = CGRP // NBUF


def _accum_node(rows, j, out_v, ln):
    def row(r, accs):
        return tuple(accs[v] + rows[j * NH + r, pl.ds(LANES * v, LANES)]
                     for v in range(VPR))

    accs = lax.fori_loop(
        1, NH, row,
        tuple(rows[j * NH, pl.ds(LANES * v, LANES)] for v in range(VPR)),
        unroll=4)
    for v in range(VPR):
        out_v[ln, pl.ds(LANES * v, LANES)] = accs[v]


def _agg_body(vx_hbm, idx_hbm, out_hbm, idx_v, rows_v, out_v, vx_sh, sems):
    sid = lax.axis_index("s")
    wid = sid * NC + lax.axis_index("c")
    base_n = wid * NPW
    base_g = wid * NGRP

    pltpu.sync_copy(vx_hbm.at[pl.ds(sid * SROWS, SROWS)],
                    vx_sh.at[pl.ds(sid * SROWS, SROWS)])

    @pl.when(sid == 0)
    def _():
        pltpu.sync_copy(vx_hbm.at[pl.ds(NS * SROWS, N - NS * SROWS)],
                        vx_sh.at[pl.ds(NS * SROWS, N - NS * SROWS)])

    pltpu.sync_copy(idx_hbm.at[pl.ds(base_g, NGRP)], idx_v)
    plsc.subcore_barrier()

    def gather(g, b):
        return pltpu.async_copy(
            vx_sh.at[idx_v.at[g]], rows_v.at[b], sems.at[b])

    for b in range(NBUF):
        gather(b, b)

    for c in range(NCHUNK):
        def wave(w, carry):
            for b in range(NBUF):
                gl = w * NBUF + b
                g = c * CGRP + gl
                pltpu.make_async_copy(
                    vx_sh.at[idx_v.at[g]], rows_v.at[b], sems.at[b]).wait()
                for j in range(GROUP):
                    _accum_node(rows_v.at[b], j, out_v, gl * GROUP + j)

                @pl.when(g + NBUF < NGRP)
                def _():
                    gather(g + NBUF, b)
            return carry

        lax.fori_loop(0, WAVES, wave, 0)

        pltpu.sync_copy(
            out_v, out_hbm.at[pl.ds(base_n + c * CGRP * GROUP,
                                    CGRP * GROUP)])


@functools.partial(
    pl.kernel,
    out_type=jax.ShapeDtypeStruct((NPAD, D), jnp.float32),
    mesh=plsc.VectorSubcoreMesh(core_axis_name="c", subcore_axis_name="s"),
    scratch_types=[
        pltpu.VMEM((NGRP, GROUP * NH), jnp.int32),
        pltpu.VMEM((NBUF, GROUP * NH, D), jnp.float32),
        pltpu.VMEM((CGRP * GROUP, D), jnp.float32),
        pltpu.VMEM_SHARED((N, D), jnp.float32),
        pltpu.SemaphoreType.DMA((NBUF,)),
    ],
)
def _aggregate(vx_hbm, idx_hbm, out_hbm, idx_v, rows_v, out_v, vx_sh, sems):
    _agg_body(vx_hbm, idx_hbm, out_hbm, idx_v, rows_v, out_v, vx_sh, sems)


def _fin_body(x_ref, a_ref, w_ref, b_ref, o_ref):
    y = (jnp.dot(x_ref[...], w_ref[:D, :],
                 preferred_element_type=jnp.float32)
         + jnp.dot(a_ref[...], w_ref[D:, :],
                   preferred_element_type=jnp.float32)
         + b_ref[...])
    o_ref[...] = jnp.maximum(y, jnp.float32(0.0))


def _finish(x, agg, w2, bias):
    blk = 1000
    return pl.pallas_call(
        _fin_body,
        grid=(N // blk,),
        in_specs=[
            pl.BlockSpec((blk, D), lambda i: (i, 0)),
            pl.BlockSpec((blk, D), lambda i: (i, 0)),
            pl.BlockSpec((2 * D, D), lambda i: (0, 0)),
            pl.BlockSpec((1, D), lambda i: (0, 0)),
        ],
        out_specs=pl.BlockSpec((blk, D), lambda i: (i, 0)),
        out_shape=jax.ShapeDtypeStruct((N, D), jnp.float32),
    )(x, agg, w2, bias.reshape(1, D))


def kernel(vertex, nh_indices, center_weight, nh_weight, bias):
    idx32 = nh_indices.astype(jnp.int32)
    idx_g = jnp.zeros((NPAD // GROUP, GROUP * NH), jnp.int32).at[
        : N // GROUP].set(idx32.reshape(N // GROUP, GROUP * NH))
    agg = _aggregate(vertex, idx_g)
    w2 = jnp.concatenate(
        [center_weight, nh_weight * jnp.float32(1.0 / NH)], axis=0)
    return _finish(vertex, agg, w2, bias)


# speedup vs baseline: 1.2148x; 1.0267x over previous
import functools

import jax
import jax.numpy as jnp
from jax import lax
from jax.experimental import pallas as pl
from jax.experimental.pallas import tpu as pltpu
from jax.experimental.pallas import tpu_sc as plsc

N = 10000
NH = 32
D = 128
LANES = 16
VPR = D // LANES

NC = 2
NS = 16
NW = NC * NS
NPAD = 10240
NPW = NPAD // NW
SROWS = 624

GROUP = 2
NGRP = NPW // GROUP
NBUF = 2
NCHUNK = 4
CGRP = NGRP // NCHUNK
WAVES = CGRP // NBUF


def _accum_node(rows, j, out_v, ln):
    def row(r, accs):
        return tuple(accs[v] + rows[j * NH + r, pl.ds(LANES * v, LANES)]
                     for v in range(VPR))

    accs = lax.fori_loop(
        1, NH, row,
        tuple(rows[j * NH, pl.ds(LANES * v, LANES)] for v in range(VPR)),
        unroll=4)
    for v in range(VPR):
        out_v[ln, pl.ds(LANES * v, LANES)] = accs[v]


def _agg_body(vx_hbm, idx_hbm, out_hbm, idx_v, rows_v, out_v, vx_sh, sems):
    sid = lax.axis_index("s")
    wid = sid * NC + lax.axis_index("c")
    base_n = wid * NPW
    base_g = wid * NGRP

    pltpu.sync_copy(vx_hbm.at[pl.ds(sid * SROWS, SROWS)],
                    vx_sh.at[pl.ds(sid * SROWS, SROWS)])

    @pl.when(sid == 0)
    def _():
        pltpu.sync_copy(vx_hbm.at[pl.ds(NS * SROWS, N - NS * SROWS)],
                        vx_sh.at[pl.ds(NS * SROWS, N - NS * SROWS)])

    pltpu.sync_copy(idx_hbm.at[pl.ds(base_g, NGRP)], idx_v)
    plsc.subcore_barrier()

    def gather(g, b):
        return pltpu.async_copy(
            vx_sh.at[idx_v.at[g]], rows_v.at[b], sems.at[b])

    for b in range(NBUF):
        gather(b, b)

    for c in range(NCHUNK):
        def wave(w, carry):
            for b in range(NBUF):
                gl = w * NBUF + b
                g = c * CGRP + gl
                pltpu.make_async_copy(
                    vx_sh.at[idx_v.at[g]], rows_v.at[b], sems.at[b]).wait()
                for j in range(GROUP):
                    _accum_node(rows_v.at[b], j, out_v, gl * GROUP + j)

                @pl.when(g + NBUF < NGRP)
                def _():
                    gather(g + NBUF, b)
            return carry

        lax.fori_loop(0, WAVES, wave, 0)

        pltpu.sync_copy(
            out_v, out_hbm.at[pl.ds(base_n + c * CGRP * GROUP,
                                    CGRP * GROUP)])


@functools.partial(
    pl.kernel,
    out_type=jax.ShapeDtypeStruct((NPAD, D), jnp.float32),
    mesh=plsc.VectorSubcoreMesh(core_axis_name="c", subcore_axis_name="s"),
    scratch_types=[
        pltpu.VMEM((NGRP, GROUP * NH), jnp.int32),
        pltpu.VMEM((NBUF, GROUP * NH, D), jnp.float32),
        pltpu.VMEM((CGRP * GROUP, D), jnp.float32),
        pltpu.VMEM_SHARED((N, D), jnp.float32),
        pltpu.SemaphoreType.DMA((NBUF,)),
    ],
)
def _aggregate(vx_hbm, idx_hbm, out_hbm, idx_v, rows_v, out_v, vx_sh, sems):
    _agg_body(vx_hbm, idx_hbm, out_hbm, idx_v, rows_v, out_v, vx_sh, sems)


def _fin_body(x_ref, a_ref, w_ref, b_ref, o_ref):
    y = (jnp.dot(x_ref[...], w_ref[:D, :],
                 preferred_element_type=jnp.float32)
         + jnp.dot(a_ref[...], w_ref[D:, :],
                   preferred_element_type=jnp.float32)
         + b_ref[...])
    o_ref[...] = jnp.maximum(y, jnp.float32(0.0))


def _finish(x, agg, w2, bias):
    blk = 2000
    return pl.pallas_call(
        _fin_body,
        grid=(N // blk,),
        in_specs=[
            pl.BlockSpec((blk, D), lambda i: (i, 0)),
            pl.BlockSpec((blk, D), lambda i: (i, 0)),
            pl.BlockSpec((2 * D, D), lambda i: (0, 0)),
            pl.BlockSpec((1, D), lambda i: (0, 0)),
        ],
        out_specs=pl.BlockSpec((blk, D), lambda i: (i, 0)),
        out_shape=jax.ShapeDtypeStruct((N, D), jnp.float32),
    )(x, agg, w2, bias.reshape(1, D))


def kernel(vertex, nh_indices, center_weight, nh_weight, bias):
    idx32 = nh_indices.astype(jnp.int32)
    idx_g = jnp.zeros((NPAD // GROUP, GROUP * NH), jnp.int32).at[
        : N // GROUP].set(idx32.reshape(N // GROUP, GROUP * NH))
    agg = _aggregate(vertex, idx_g)
    w2 = jnp.concatenate(
        [center_weight, nh_weight * jnp.float32(1.0 / NH)], axis=0)
    return _finish(vertex, agg, w2, bias)


# speedup vs baseline: 1.2247x; 1.0082x over previous
import functools

import jax
import jax.numpy as jnp
from jax import lax
from jax.experimental import pallas as pl
from jax.experimental.pallas import tpu as pltpu
from jax.experimental.pallas import tpu_sc as plsc

N = 10000
NH = 32
D = 128
LANES = 16
VPR = D // LANES

NC = 2
NS = 16
NW = NC * NS
NPAD = 10240
NPW = NPAD // NW
SROWS = 624

GROUP = 2
NGRP = NPW // GROUP
NBUF = 2
NCHUNK = 4
CGRP = NGRP // NCHUNK
WAVES = CGRP // NBUF


def _accum_node(rows, j, out_v, ln):
    def row(r, accs):
        return tuple(accs[v] + rows[j * NH + r, pl.ds(LANES * v, LANES)]
                     for v in range(VPR))

    accs = lax.fori_loop(
        1, NH, row,
        tuple(rows[j * NH, pl.ds(LANES * v, LANES)] for v in range(VPR)),
        unroll=2)
    for v in range(VPR):
        out_v[ln, pl.ds(LANES * v, LANES)] = accs[v]


def _agg_body(vx_hbm, idx_hbm, out_hbm, idx_v, rows_v, out_v, vx_sh, sems):
    sid = lax.axis_index("s")
    wid = sid * NC + lax.axis_index("c")
    base_n = wid * NPW
    base_g = wid * NGRP

    pltpu.sync_copy(vx_hbm.at[pl.ds(sid * SROWS, SROWS)],
                    vx_sh.at[pl.ds(sid * SROWS, SROWS)])

    @pl.when(sid == 0)
    def _():
        pltpu.sync_copy(vx_hbm.at[pl.ds(NS * SROWS, N - NS * SROWS)],
                        vx_sh.at[pl.ds(NS * SROWS, N - NS * SROWS)])

    pltpu.sync_copy(idx_hbm.at[pl.ds(base_g, NGRP)], idx_v)
    plsc.subcore_barrier()

    def gather(g, b):
        return pltpu.async_copy(
            vx_sh.at[idx_v.at[g]], rows_v.at[b], sems.at[b])

    for b in range(NBUF):
        gather(b, b)

    for c in range(NCHUNK):
        def wave(w, carry):
            for b in range(NBUF):
                gl = w * NBUF + b
                g = c * CGRP + gl
                pltpu.make_async_copy(
                    vx_sh.at[idx_v.at[g]], rows_v.at[b], sems.at[b]).wait()
                for j in range(GROUP):
                    _accum_node(rows_v.at[b], j, out_v, gl * GROUP + j)

                @pl.when(g + NBUF < NGRP)
                def _():
                    gather(g + NBUF, b)
            return carry

        lax.fori_loop(0, WAVES, wave, 0)

        pltpu.sync_copy(
            out_v, out_hbm.at[pl.ds(base_n + c * CGRP * GROUP,
                                    CGRP * GROUP)])


@functools.partial(
    pl.kernel,
    out_type=jax.ShapeDtypeStruct((NPAD, D), jnp.float32),
    mesh=plsc.VectorSubcoreMesh(core_axis_name="c", subcore_axis_name="s"),
    scratch_types=[
        pltpu.VMEM((NGRP, GROUP * NH), jnp.int32),
        pltpu.VMEM((NBUF, GROUP * NH, D), jnp.float32),
        pltpu.VMEM((CGRP * GROUP, D), jnp.float32),
        pltpu.VMEM_SHARED((N, D), jnp.float32),
        pltpu.SemaphoreType.DMA((NBUF,)),
    ],
)
def _aggregate(vx_hbm, idx_hbm, out_hbm, idx_v, rows_v, out_v, vx_sh, sems):
    _agg_body(vx_hbm, idx_hbm, out_hbm, idx_v, rows_v, out_v, vx_sh, sems)


def _fin_body(x_ref, a_ref, w_ref, b_ref, o_ref):
    y = (jnp.dot(x_ref[...], w_ref[:D, :],
                 preferred_element_type=jnp.float32)
         + jnp.dot(a_ref[...], w_ref[D:, :],
                   preferred_element_type=jnp.float32)
         + b_ref[...])
    o_ref[...] = jnp.maximum(y, jnp.float32(0.0))


def _finish(x, agg, w2, bias):
    blk = 2000
    return pl.pallas_call(
        _fin_body,
        grid=(N // blk,),
        in_specs=[
            pl.BlockSpec((blk, D), lambda i: (i, 0)),
            pl.BlockSpec((blk, D), lambda i: (i, 0)),
            pl.BlockSpec((2 * D, D), lambda i: (0, 0)),
            pl.BlockSpec((1, D), lambda i: (0, 0)),
        ],
        out_specs=pl.BlockSpec((blk, D), lambda i: (i, 0)),
        out_shape=jax.ShapeDtypeStruct((N, D), jnp.float32),
    )(x, agg, w2, bias.reshape(1, D))


def kernel(vertex, nh_indices, center_weight, nh_weight, bias):
    idx32 = nh_indices.astype(jnp.int32)
    idx_g = jnp.zeros((NPAD // GROUP, GROUP * NH), jnp.int32).at[
        : N // GROUP].set(idx32.reshape(N // GROUP, GROUP * NH))
    agg = _aggregate(vertex, idx_g)
    w2 = jnp.concatenate(
        [center_weight, nh_weight * jnp.float32(1.0 / NH)], axis=0)
    return _finish(vertex, agg, w2, bias)


# speedup vs baseline: 1.2269x; 1.0018x over previous
import functools

import jax
import jax.numpy as jnp
from jax import lax
from jax.experimental import pallas as pl
from jax.experimental.pallas import tpu as pltpu
from jax.experimental.pallas import tpu_sc as plsc

N = 10000
NH = 32
D = 128
LANES = 16
VPR = D // LANES

NC = 2
NS = 16
NW = NC * NS
NPAD = 10240
NPW = NPAD // NW
SROWS = 624

GROUP = 2
NGRP = NPW // GROUP
NBUF = 2
NCHUNK = 4
CGRP = NGRP // NCHUNK
WAVES = CGRP // NBUF


def _accum_node(rows, j, out_v, ln):
    def row(r, accs):
        return tuple(accs[v] + rows[j * NH + r, pl.ds(LANES * v, LANES)]
                     for v in range(VPR))

    accs = lax.fori_loop(
        1, NH, row,
        tuple(rows[j * NH, pl.ds(LANES * v, LANES)] for v in range(VPR)),
        unroll=1)
    for v in range(VPR):
        out_v[ln, pl.ds(LANES * v, LANES)] = accs[v]


def _agg_body(vx_hbm, idx_hbm, out_hbm, idx_v, rows_v, out_v, vx_sh, sems):
    sid = lax.axis_index("s")
    wid = sid * NC + lax.axis_index("c")
    base_n = wid * NPW
    base_g = wid * NGRP

    pltpu.sync_copy(vx_hbm.at[pl.ds(sid * SROWS, SROWS)],
                    vx_sh.at[pl.ds(sid * SROWS, SROWS)])

    @pl.when(sid == 0)
    def _():
        pltpu.sync_copy(vx_hbm.at[pl.ds(NS * SROWS, N - NS * SROWS)],
                        vx_sh.at[pl.ds(NS * SROWS, N - NS * SROWS)])

    pltpu.sync_copy(idx_hbm.at[pl.ds(base_g, NGRP)], idx_v)
    plsc.subcore_barrier()

    def gather(g, b):
        return pltpu.async_copy(
            vx_sh.at[idx_v.at[g]], rows_v.at[b], sems.at[b])

    for b in range(NBUF):
        gather(b, b)

    for c in range(NCHUNK):
        def wave(w, carry):
            for b in range(NBUF):
                gl = w * NBUF + b
                g = c * CGRP + gl
                pltpu.make_async_copy(
                    vx_sh.at[idx_v.at[g]], rows_v.at[b], sems.at[b]).wait()
                for j in range(GROUP):
                    _accum_node(rows_v.at[b], j, out_v, gl * GROUP + j)

                @pl.when(g + NBUF < NGRP)
                def _():
                    gather(g + NBUF, b)
            return carry

        lax.fori_loop(0, WAVES, wave, 0)

        pltpu.sync_copy(
            out_v, out_hbm.at[pl.ds(base_n + c * CGRP * GROUP,
                                    CGRP * GROUP)])


@functools.partial(
    pl.kernel,
    out_type=jax.ShapeDtypeStruct((NPAD, D), jnp.float32),
    mesh=plsc.VectorSubcoreMesh(core_axis_name="c", subcore_axis_name="s"),
    scratch_types=[
        pltpu.VMEM((NGRP, GROUP * NH), jnp.int32),
        pltpu.VMEM((NBUF, GROUP * NH, D), jnp.float32),
        pltpu.VMEM((CGRP * GROUP, D), jnp.float32),
        pltpu.VMEM_SHARED((N, D), jnp.float32),
        pltpu.SemaphoreType.DMA((NBUF,)),
    ],
)
def _aggregate(vx_hbm, idx_hbm, out_hbm, idx_v, rows_v, out_v, vx_sh, sems):
    _agg_body(vx_hbm, idx_hbm, out_hbm, idx_v, rows_v, out_v, vx_sh, sems)


def _fin_body(x_ref, a_ref, w_ref, b_ref, o_ref):
    y = (jnp.dot(x_ref[...], w_ref[:D, :],
                 preferred_element_type=jnp.float32)
         + jnp.dot(a_ref[...], w_ref[D:, :],
                   preferred_element_type=jnp.float32)
         + b_ref[...])
    o_ref[...] = jnp.maximum(y, jnp.float32(0.0))


def _finish(x, agg, w2, bias):
    blk = 2000
    return pl.pallas_call(
        _fin_body,
        grid=(N // blk,),
        in_specs=[
            pl.BlockSpec((blk, D), lambda i: (i, 0)),
            pl.BlockSpec((blk, D), lambda i: (i, 0)),
            pl.BlockSpec((2 * D, D), lambda i: (0, 0)),
            pl.BlockSpec((1, D), lambda i: (0, 0)),
        ],
        out_specs=pl.BlockSpec((blk, D), lambda i: (i, 0)),
        out_shape=jax.ShapeDtypeStruct((N, D), jnp.float32),
    )(x, agg, w2, bias.reshape(1, D))


def kernel(vertex, nh_indices, center_weight, nh_weight, bias):
    idx32 = nh_indices.astype(jnp.int32)
    idx_g = jnp.zeros((NPAD // GROUP, GROUP * NH), jnp.int32).at[
        : N // GROUP].set(idx32.reshape(N // GROUP, GROUP * NH))
    agg = _aggregate(vertex, idx_g)
    w2 = jnp.concatenate(
        [center_weight, nh_weight * jnp.float32(1.0 / NH)], axis=0)
    return _finish(vertex, agg, w2, bias)


# speedup vs baseline: 1.2487x; 1.0178x over previous
import functools

import jax
import jax.numpy as jnp
from jax import lax
from jax.experimental import pallas as pl
from jax.experimental.pallas import tpu as pltpu
from jax.experimental.pallas import tpu_sc as plsc

N = 10000
NH = 32
D = 128
LANES = 16
VPR = D // LANES

NC = 2
NS = 16
NW = NC * NS
NPAD = 10240
NPW = NPAD // NW
SROWS = 624

GROUP = 4
NGRP = NPW // GROUP
NBUF = 2
NCHUNK = 8
CGRP = NGRP // NCHUNK
WAVES = CGRP // NBUF


def _accum_node(rows, j, out_v, ln):
    def row(r, accs):
        return tuple(accs[v] + rows[j * NH + r, pl.ds(LANES * v, LANES)]
                     for v in range(VPR))

    accs = lax.fori_loop(
        1, NH, row,
        tuple(rows[j * NH, pl.ds(LANES * v, LANES)] for v in range(VPR)),
        unroll=1)
    for v in range(VPR):
        out_v[ln, pl.ds(LANES * v, LANES)] = accs[v]


def _agg_body(vx_hbm, idx_hbm, out_hbm, idx_v, rows_v, out_v, vx_sh, sems):
    sid = lax.axis_index("s")
    wid = sid * NC + lax.axis_index("c")
    base_n = wid * NPW
    base_g = wid * NGRP

    pltpu.sync_copy(vx_hbm.at[pl.ds(sid * SROWS, SROWS)],
                    vx_sh.at[pl.ds(sid * SROWS, SROWS)])

    @pl.when(sid == 0)
    def _():
        pltpu.sync_copy(vx_hbm.at[pl.ds(NS * SROWS, N - NS * SROWS)],
                        vx_sh.at[pl.ds(NS * SROWS, N - NS * SROWS)])

    pltpu.sync_copy(idx_hbm.at[pl.ds(base_g, NGRP)], idx_v)
    plsc.subcore_barrier()

    def gather(g, b):
        return pltpu.async_copy(
            vx_sh.at[idx_v.at[g]], rows_v.at[b], sems.at[b])

    for b in range(NBUF):
        gather(b, b)

    for c in range(NCHUNK):
        def wave(w, carry):
            for b in range(NBUF):
                gl = w * NBUF + b
                g = c * CGRP + gl
                pltpu.make_async_copy(
                    vx_sh.at[idx_v.at[g]], rows_v.at[b], sems.at[b]).wait()
                for j in range(GROUP):
                    _accum_node(rows_v.at[b], j, out_v, gl * GROUP + j)

                @pl.when(g + NBUF < NGRP)
                def _():
                    gather(g + NBUF, b)
            return carry

        lax.fori_loop(0, WAVES, wave, 0)

        pltpu.sync_copy(
            out_v, out_hbm.at[pl.ds(base_n + c * CGRP * GROUP,
                                    CGRP * GROUP)])


@functools.partial(
    pl.kernel,
    out_type=jax.ShapeDtypeStruct((NPAD, D), jnp.float32),
    mesh=plsc.VectorSubcoreMesh(core_axis_name="c", subcore_axis_name="s"),
    scratch_types=[
        pltpu.VMEM((NGRP, GROUP * NH), jnp.int32),
        pltpu.VMEM((NBUF, GROUP * NH, D), jnp.float32),
        pltpu.VMEM((CGRP * GROUP, D), jnp.float32),
        pltpu.VMEM_SHARED((N, D), jnp.float32),
        pltpu.SemaphoreType.DMA((NBUF,)),
    ],
)
def _aggregate(vx_hbm, idx_hbm, out_hbm, idx_v, rows_v, out_v, vx_sh, sems):
    _agg_body(vx_hbm, idx_hbm, out_hbm, idx_v, rows_v, out_v, vx_sh, sems)


def _fin_body(x_ref, a_ref, w_ref, b_ref, o_ref):
    y = (jnp.dot(x_ref[...], w_ref[:D, :],
                 preferred_element_type=jnp.float32)
         + jnp.dot(a_ref[...], w_ref[D:, :],
                   preferred_element_type=jnp.float32)
         + b_ref[...])
    o_ref[...] = jnp.maximum(y, jnp.float32(0.0))


def _finish(x, agg, w2, bias):
    blk = 2000
    return pl.pallas_call(
        _fin_body,
        grid=(N // blk,),
        in_specs=[
            pl.BlockSpec((blk, D), lambda i: (i, 0)),
            pl.BlockSpec((blk, D), lambda i: (i, 0)),
            pl.BlockSpec((2 * D, D), lambda i: (0, 0)),
            pl.BlockSpec((1, D), lambda i: (0, 0)),
        ],
        out_specs=pl.BlockSpec((blk, D), lambda i: (i, 0)),
        out_shape=jax.ShapeDtypeStruct((N, D), jnp.float32),
    )(x, agg, w2, bias.reshape(1, D))


def kernel(vertex, nh_indices, center_weight, nh_weight, bias):
    idx32 = nh_indices.astype(jnp.int32)
    idx_g = jnp.zeros((NPAD // GROUP, GROUP * NH), jnp.int32).at[
        : N // GROUP].set(idx32.reshape(N // GROUP, GROUP * NH))
    agg = _aggregate(vertex, idx_g)
    w2 = jnp.concatenate(
        [center_weight, nh_weight * jnp.float32(1.0 / NH)], axis=0)
    return _finish(vertex, agg, w2, bias)


# speedup vs baseline: 1.2686x; 1.0159x over previous
import functools

import jax
import jax.numpy as jnp
from jax import lax
from jax.experimental import pallas as pl
from jax.experimental.pallas import tpu as pltpu
from jax.experimental.pallas import tpu_sc as plsc

N = 10000
NH = 32
D = 128
LANES = 16
VPR = D // LANES

NC = 2
NS = 16
NW = NC * NS
NPAD = 10240
NPW = NPAD // NW
SROWS = 624

GROUP = 4
NGRP = NPW // GROUP
NBUF = 2
NCHUNK = 8
CGRP = NGRP // NCHUNK
WAVES = CGRP // NBUF


def _accum_node(rows, j, out_v, ln):
    def row(r, accs):
        return tuple(accs[v] + rows[j * NH + r, pl.ds(LANES * v, LANES)]
                     for v in range(VPR))

    accs = lax.fori_loop(
        1, NH, row,
        tuple(rows[j * NH, pl.ds(LANES * v, LANES)] for v in range(VPR)),
        unroll=1)
    for v in range(VPR):
        out_v[ln, pl.ds(LANES * v, LANES)] = accs[v]


def _agg_body(vx_hbm, idx_hbm, out_hbm, idx_v, rows_v, out_v, vx_sh, sems):
    sid = lax.axis_index("s")
    wid = sid * NC + lax.axis_index("c")
    base_n = wid * NPW
    base_g = wid * NGRP

    pltpu.sync_copy(vx_hbm.at[pl.ds(sid * SROWS, SROWS)],
                    vx_sh.at[pl.ds(sid * SROWS, SROWS)])

    @pl.when(sid == 0)
    def _():
        pltpu.sync_copy(vx_hbm.at[pl.ds(NS * SROWS, N - NS * SROWS)],
                        vx_sh.at[pl.ds(NS * SROWS, N - NS * SROWS)])

    pltpu.sync_copy(idx_hbm.at[pl.ds(base_g, NGRP)], idx_v)
    plsc.subcore_barrier()

    def gather(g, b):
        return pltpu.async_copy(
            vx_sh.at[idx_v.at[g]], rows_v.at[b], sems.at[b])

    for b in range(NBUF):
        gather(b, b)

    for c in range(NCHUNK):
        def wave(w, carry):
            for b in range(NBUF):
                gl = w * NBUF + b
                g = c * CGRP + gl
                pltpu.make_async_copy(
                    vx_sh.at[idx_v.at[g]], rows_v.at[b], sems.at[b]).wait()
                for j in range(GROUP):
                    _accum_node(rows_v.at[b], j, out_v, gl * GROUP + j)

                @pl.when(g + NBUF < NGRP)
                def _():
                    gather(g + NBUF, b)
            return carry

        lax.fori_loop(0, WAVES, wave, 0)

        pltpu.sync_copy(
            out_v, out_hbm.at[pl.ds(base_n + c * CGRP * GROUP,
                                    CGRP * GROUP)])


@functools.partial(
    pl.kernel,
    out_type=jax.ShapeDtypeStruct((NPAD, D), jnp.float32),
    mesh=plsc.VectorSubcoreMesh(core_axis_name="c", subcore_axis_name="s"),
    scratch_types=[
        pltpu.VMEM((NGRP, GROUP * NH), jnp.int32),
        pltpu.VMEM((NBUF, GROUP * NH, D), jnp.float32),
        pltpu.VMEM((CGRP * GROUP, D), jnp.float32),
        pltpu.VMEM_SHARED((N, D), jnp.float32),
        pltpu.SemaphoreType.DMA((NBUF,)),
    ],
)
def _aggregate(vx_hbm, idx_hbm, out_hbm, idx_v, rows_v, out_v, vx_sh, sems):
    _agg_body(vx_hbm, idx_hbm, out_hbm, idx_v, rows_v, out_v, vx_sh, sems)


def _fin_body(x_ref, a_ref, w_ref, b_ref, o_ref):
    y = (jnp.dot(x_ref[...], w_ref[:D, :],
                 preferred_element_type=jnp.float32)
         + jnp.dot(a_ref[...], w_ref[D:, :],
                   preferred_element_type=jnp.float32)
         + b_ref[...])
    o_ref[...] = jnp.maximum(y, jnp.float32(0.0))


def _finish(x, agg, w2, bias):
    blk = 5000
    return pl.pallas_call(
        _fin_body,
        grid=(N // blk,),
        in_specs=[
            pl.BlockSpec((blk, D), lambda i: (i, 0)),
            pl.BlockSpec((blk, D), lambda i: (i, 0)),
            pl.BlockSpec((2 * D, D), lambda i: (0, 0)),
            pl.BlockSpec((1, D), lambda i: (0, 0)),
        ],
        out_specs=pl.BlockSpec((blk, D), lambda i: (i, 0)),
        out_shape=jax.ShapeDtypeStruct((N, D), jnp.float32),
    )(x, agg, w2, bias.reshape(1, D))


def kernel(vertex, nh_indices, center_weight, nh_weight, bias):
    idx32 = nh_indices.astype(jnp.int32)
    idx_g = jnp.zeros((NPAD // GROUP, GROUP * NH), jnp.int32).at[
        : N // GROUP].set(idx32.reshape(N // GROUP, GROUP * NH))
    agg = _aggregate(vertex, idx_g)
    w2 = jnp.concatenate(
        [center_weight, nh_weight * jnp.float32(1.0 / NH)], axis=0)
    return _finish(vertex, agg, w2, bias)
